# baseline (speedup 1.0000x reference)
import functools

import jax
import jax.numpy as jnp
from jax import lax
from jax.experimental import pallas as pl
from jax.experimental.pallas import tpu as pltpu
from jax.experimental.pallas import tpu_sc as plsc

_VOCAB = 100000
_EMBED = 300
_LANES = 16
_NC = 2
_NS = 16
_NW = _NC * _NS

_SC_COLS = 49152
_SC_COLS_W = _SC_COLS // _NW
_SC_CHUNK = 128
_TC_COLS = _VOCAB - _SC_COLS
_COLS_BLK = 8192
_TC_BLK0 = _SC_COLS // _COLS_BLK

_CHUNK = 6400


def _proj_body(t_ref, w_ref, b_ref, o_ref):
    t = t_ref[...]
    w = w_ref[...]
    z = jnp.sum(t * w, axis=0, keepdims=True) + b_ref[0, 0]
    o_ref[...] = jax.nn.sigmoid(jnp.maximum(z, 0.0))


def _project_table_tc(tT, W, b):
    return pl.pallas_call(
        _proj_body,
        grid=(pl.cdiv(_TC_COLS, _COLS_BLK),),
        in_specs=[
            pl.BlockSpec((_EMBED, _COLS_BLK), lambda i: (0, i + _TC_BLK0)),
            pl.BlockSpec((_EMBED, 1), lambda i: (0, 0)),
            pl.BlockSpec((1, 1), lambda i: (0, 0)),
        ],
        out_specs=pl.BlockSpec((1, _COLS_BLK), lambda i: (0, i)),
        out_shape=jax.ShapeDtypeStruct((1, _TC_COLS), jnp.float32),
    )(tT, W.reshape(_EMBED, 1), b.reshape(1, 1))


def _make_sc_proj():
    mesh = plsc.VectorSubcoreMesh(core_axis_name="c", subcore_axis_name="s")
    n_groups = _SC_CHUNK // _LANES

    @functools.partial(
        pl.kernel,
        mesh=mesh,
        out_type=jax.ShapeDtypeStruct((_SC_COLS,), jnp.float32),
        scratch_types=[
            pltpu.VMEM((_EMBED, _SC_CHUNK), jnp.float32),
            pltpu.VMEM((_EMBED, _SC_CHUNK), jnp.float32),
            pltpu.VMEM((_EMBED,), jnp.float32),
            pltpu.VMEM((_LANES,), jnp.float32),
            pltpu.VMEM((_SC_COLS_W,), jnp.float32),
            pltpu.SemaphoreType.DMA,
            pltpu.SemaphoreType.DMA,
        ],
        compiler_params=pltpu.CompilerParams(needs_layout_passes=False),
    )
    def sc_proj_k(tT_hbm, w_hbm, b_hbm, s_hbm, buf0, buf1, w_v, b_v, out_v,
                  sem_b0, sem_b1):
        bufs = (buf0, buf1)
        wid = lax.axis_index("s") * _NC + lax.axis_index("c")
        col0 = wid * _SC_COLS_W
        n_chunks = _SC_COLS_W // _SC_CHUNK
        sem_b = (sem_b0, sem_b1)
        pltpu.sync_copy(w_hbm, w_v)
        pltpu.sync_copy(b_hbm, b_v)
        b_vec = b_v[...]
        cp = {}
        for c in range(min(2, n_chunks)):
            cp[c] = pltpu.async_copy(
                tT_hbm.at[:, pl.ds(col0 + c * _SC_CHUNK, _SC_CHUNK)],
                bufs[c % 2], sem_b[c % 2])
        for c in range(n_chunks):
            sl = c % 2
            buf = bufs[sl]
            cp[c].wait()

            def body(d, accs):
                wb = plsc.load_gather(
                    w_v, [jnp.full((_LANES,), d, jnp.int32)])
                return tuple(
                    a + buf[d, pl.ds(g * _LANES, _LANES)] * wb
                    for g, a in enumerate(accs))

            accs = lax.fori_loop(
                0, _EMBED, body,
                tuple(jnp.zeros((_LANES,), jnp.float32)
                      for _ in range(n_groups)))
            for g in range(n_groups):
                z = jnp.maximum(accs[g] + b_vec, 0.0)
                out_v[pl.ds(c * _SC_CHUNK + g * _LANES, _LANES)] = (
                    1.0 / (1.0 + jnp.exp(-z)))
            if c + 2 < n_chunks:
                cp[c + 2] = pltpu.async_copy(
                    tT_hbm.at[:, pl.ds(col0 + (c + 2) * _SC_CHUNK,
                                       _SC_CHUNK)],
                    bufs[sl], sem_b[sl])
        pltpu.sync_copy(out_v, s_hbm.at[pl.ds(col0, _SC_COLS_W)])

    return sc_proj_k


@functools.lru_cache(maxsize=None)
def _make_gather(total):
    per_w = total // _NW
    n_chunks = per_w // _CHUNK
    mesh = plsc.VectorSubcoreMesh(core_axis_name="c", subcore_axis_name="s")

    @functools.partial(
        pl.kernel,
        mesh=mesh,
        out_type=jax.ShapeDtypeStruct((total,), jnp.float32),
        scratch_types=[
            pltpu.VMEM((_VOCAB,), jnp.float32),
            pltpu.VMEM((2, _CHUNK), jnp.int32),
            pltpu.VMEM((2, _CHUNK), jnp.float32),
            pltpu.SemaphoreType.DMA,
            pltpu.SemaphoreType.DMA,
            pltpu.SemaphoreType.DMA,
            pltpu.SemaphoreType.DMA,
            pltpu.SemaphoreType.DMA,
            pltpu.SemaphoreType.DMA,
        ],
        compiler_params=pltpu.CompilerParams(needs_layout_passes=False),
    )
    def gather_k(s_sc_hbm, s_tc_hbm, idx_hbm, out_hbm, s_v, idx_v, out_v,
                 sem_s0, sem_s1, sem_i0, sem_i1, sem_o0, sem_o1):
        wid = lax.axis_index("s") * _NC + lax.axis_index("c")
        base = wid * per_w
        sem_i = (sem_i0, sem_i1)
        sem_o = (sem_o0, sem_o1)

        cp_s0 = pltpu.async_copy(
            s_sc_hbm, s_v.at[pl.ds(0, _SC_COLS)], sem_s0)
        cp_s1 = pltpu.async_copy(
            s_tc_hbm.at[0], s_v.at[pl.ds(_SC_COLS, _TC_COLS)], sem_s1)
        cp_i = {}
        for c in range(min(2, n_chunks)):
            cp_i[c] = pltpu.async_copy(
                idx_hbm.at[pl.ds(base + c * _CHUNK, _CHUNK)],
                idx_v.at[c % 2], sem_i[c % 2])
        cp_s0.wait()
        cp_s1.wait()

        cp_o = {}
        for c in range(n_chunks):
            sl = c % 2
            cp_i[c].wait()
            if c >= 2:
                cp_o[c - 2].wait()

            @plsc.parallel_loop(0, _CHUNK // _LANES, unroll=8)
            def _(i):
                idx16 = idx_v[sl, pl.ds(i * _LANES, _LANES)]
                out_v[sl, pl.ds(i * _LANES, _LANES)] = plsc.load_gather(
                    s_v, [idx16])

            cp_o[c] = pltpu.async_copy(
                out_v.at[sl], out_hbm.at[pl.ds(base + c * _CHUNK, _CHUNK)],
                sem_o[sl])
            if c + 2 < n_chunks:
                cp_i[c + 2] = pltpu.async_copy(
                    idx_hbm.at[pl.ds(base + (c + 2) * _CHUNK, _CHUNK)],
                    idx_v.at[sl], sem_i[sl])

        for c in range(max(0, n_chunks - 2), n_chunks):
            cp_o[c].wait()

    return gather_k


def kernel(x, table, W, b):
    tT = table.T
    s_sc = _make_sc_proj()(tT, W.reshape(_EMBED), jnp.broadcast_to(b, (16,)))
    s_tc = _project_table_tc(tT, W, b)
    xf = x.reshape(-1)
    out = _make_gather(xf.size)(s_sc, s_tc, xf)
    return out.reshape(-1, 1)


# speedup vs baseline: 1.0177x; 1.0177x over previous
import functools

import jax
import jax.numpy as jnp
from jax import lax
from jax.experimental import pallas as pl
from jax.experimental.pallas import tpu as pltpu
from jax.experimental.pallas import tpu_sc as plsc

_VOCAB = 100000
_EMBED = 300
_LANES = 16
_NC = 2
_NS = 16
_NW = _NC * _NS

_SC_COLS = 36864
_SC_COLS_W = _SC_COLS // _NW
_SC_CHUNK = 128
_TC_COLS = _VOCAB - _SC_COLS
_COLS_BLK = 9216
_TC_BLK0 = _SC_COLS // _COLS_BLK

_CHUNK = 6400


def _proj_body(t_ref, w_ref, b_ref, o_ref):
    t = t_ref[...]
    w = w_ref[...]
    z = jnp.sum(t * w, axis=0, keepdims=True) + b_ref[0, 0]
    o_ref[...] = jax.nn.sigmoid(jnp.maximum(z, 0.0))


def _project_table_tc(tT, W, b):
    return pl.pallas_call(
        _proj_body,
        grid=(pl.cdiv(_TC_COLS, _COLS_BLK),),
        in_specs=[
            pl.BlockSpec((_EMBED, _COLS_BLK), lambda i: (0, i + _TC_BLK0)),
            pl.BlockSpec((_EMBED, 1), lambda i: (0, 0)),
            pl.BlockSpec((1, 1), lambda i: (0, 0)),
        ],
        out_specs=pl.BlockSpec((1, _COLS_BLK), lambda i: (0, i)),
        out_shape=jax.ShapeDtypeStruct((1, _TC_COLS), jnp.float32),
    )(tT, W.reshape(_EMBED, 1), b.reshape(1, 1))


def _make_sc_proj():
    mesh = plsc.VectorSubcoreMesh(core_axis_name="c", subcore_axis_name="s")
    n_groups = _SC_CHUNK // _LANES

    @functools.partial(
        pl.kernel,
        mesh=mesh,
        out_type=jax.ShapeDtypeStruct((_SC_COLS,), jnp.float32),
        scratch_types=[
            pltpu.VMEM((_EMBED, _SC_CHUNK), jnp.float32),
            pltpu.VMEM((_EMBED, _SC_CHUNK), jnp.float32),
            pltpu.VMEM((_EMBED,), jnp.float32),
            pltpu.VMEM((_LANES,), jnp.float32),
            pltpu.VMEM((_SC_COLS_W,), jnp.float32),
            pltpu.SemaphoreType.DMA,
            pltpu.SemaphoreType.DMA,
        ],
        compiler_params=pltpu.CompilerParams(needs_layout_passes=False),
    )
    def sc_proj_k(tT_hbm, w_hbm, b_hbm, s_hbm, buf0, buf1, w_v, b_v, out_v,
                  sem_b0, sem_b1):
        bufs = (buf0, buf1)
        wid = lax.axis_index("s") * _NC + lax.axis_index("c")
        col0 = wid * _SC_COLS_W
        n_chunks = _SC_COLS_W // _SC_CHUNK
        sem_b = (sem_b0, sem_b1)
        pltpu.sync_copy(w_hbm, w_v)
        pltpu.sync_copy(b_hbm, b_v)
        b_vec = b_v[...]
        cp = {}
        for c in range(min(2, n_chunks)):
            cp[c] = pltpu.async_copy(
                tT_hbm.at[:, pl.ds(col0 + c * _SC_CHUNK, _SC_CHUNK)],
                bufs[c % 2], sem_b[c % 2])
        for c in range(n_chunks):
            sl = c % 2
            buf = bufs[sl]
            cp[c].wait()

            def body(d, accs):
                wb = plsc.load_gather(
                    w_v, [jnp.full((_LANES,), d, jnp.int32)])
                return tuple(
                    a + buf[d, pl.ds(g * _LANES, _LANES)] * wb
                    for g, a in enumerate(accs))

            accs = lax.fori_loop(
                0, _EMBED, body,
                tuple(jnp.zeros((_LANES,), jnp.float32)
                      for _ in range(n_groups)))
            for g in range(n_groups):
                z = jnp.maximum(accs[g] + b_vec, 0.0)
                out_v[pl.ds(c * _SC_CHUNK + g * _LANES, _LANES)] = (
                    1.0 / (1.0 + jnp.exp(-z)))
            if c + 2 < n_chunks:
                cp[c + 2] = pltpu.async_copy(
                    tT_hbm.at[:, pl.ds(col0 + (c + 2) * _SC_CHUNK,
                                       _SC_CHUNK)],
                    bufs[sl], sem_b[sl])
        pltpu.sync_copy(out_v, s_hbm.at[pl.ds(col0, _SC_COLS_W)])

    return sc_proj_k


@functools.lru_cache(maxsize=None)
def _make_gather(total):
    per_w = total // _NW
    n_chunks = per_w // _CHUNK
    mesh = plsc.VectorSubcoreMesh(core_axis_name="c", subcore_axis_name="s")

    @functools.partial(
        pl.kernel,
        mesh=mesh,
        out_type=jax.ShapeDtypeStruct((total,), jnp.float32),
        scratch_types=[
            pltpu.VMEM((_VOCAB,), jnp.float32),
            pltpu.VMEM((2, _CHUNK), jnp.int32),
            pltpu.VMEM((2, _CHUNK), jnp.float32),
            pltpu.SemaphoreType.DMA,
            pltpu.SemaphoreType.DMA,
            pltpu.SemaphoreType.DMA,
            pltpu.SemaphoreType.DMA,
            pltpu.SemaphoreType.DMA,
            pltpu.SemaphoreType.DMA,
        ],
        compiler_params=pltpu.CompilerParams(needs_layout_passes=False),
    )
    def gather_k(s_sc_hbm, s_tc_hbm, idx_hbm, out_hbm, s_v, idx_v, out_v,
                 sem_s0, sem_s1, sem_i0, sem_i1, sem_o0, sem_o1):
        wid = lax.axis_index("s") * _NC + lax.axis_index("c")
        base = wid * per_w
        sem_i = (sem_i0, sem_i1)
        sem_o = (sem_o0, sem_o1)

        cp_s0 = pltpu.async_copy(
            s_sc_hbm, s_v.at[pl.ds(0, _SC_COLS)], sem_s0)
        cp_s1 = pltpu.async_copy(
            s_tc_hbm.at[0], s_v.at[pl.ds(_SC_COLS, _TC_COLS)], sem_s1)
        cp_i = {}
        for c in range(min(2, n_chunks)):
            cp_i[c] = pltpu.async_copy(
                idx_hbm.at[pl.ds(base + c * _CHUNK, _CHUNK)],
                idx_v.at[c % 2], sem_i[c % 2])
        cp_s0.wait()
        cp_s1.wait()

        cp_o = {}
        for c in range(n_chunks):
            sl = c % 2
            cp_i[c].wait()
            if c >= 2:
                cp_o[c - 2].wait()

            @plsc.parallel_loop(0, _CHUNK // _LANES, unroll=8)
            def _(i):
                idx16 = idx_v[sl, pl.ds(i * _LANES, _LANES)]
                out_v[sl, pl.ds(i * _LANES, _LANES)] = plsc.load_gather(
                    s_v, [idx16])

            cp_o[c] = pltpu.async_copy(
                out_v.at[sl], out_hbm.at[pl.ds(base + c * _CHUNK, _CHUNK)],
                sem_o[sl])
            if c + 2 < n_chunks:
                cp_i[c + 2] = pltpu.async_copy(
                    idx_hbm.at[pl.ds(base + (c + 2) * _CHUNK, _CHUNK)],
                    idx_v.at[sl], sem_i[sl])

        for c in range(max(0, n_chunks - 2), n_chunks):
            cp_o[c].wait()

    return gather_k


def kernel(x, table, W, b):
    tT = table.T
    s_sc = _make_sc_proj()(tT, W.reshape(_EMBED), jnp.broadcast_to(b, (16,)))
    s_tc = _project_table_tc(tT, W, b)
    xf = x.reshape(-1)
    out = _make_gather(xf.size)(s_sc, s_tc, xf)
    return out.reshape(-1, 1)
